# Initial kernel scaffold; baseline (speedup 1.0000x reference)
import jax
import jax.numpy as jnp
from jax.experimental import pallas as pl


def kernel(idx, log_probs):
    raise NotImplementedError("write your pallas kernel here")


# speedup vs baseline: 1.1298x; 1.1298x over previous
import functools

import jax
import jax.numpy as jnp
from jax import lax
from jax.experimental import pallas as pl
from jax.experimental.pallas import tpu as pltpu
from jax.experimental.pallas import tpu_sc as plsc

_NC = 2
_NS = 16
_NW = _NC * _NS
_CHUNK = 40


@functools.lru_cache(maxsize=None)
def _build(R, V, dtype_name):
    dtype = jnp.dtype(dtype_name)
    RPW = R // _NW
    NCH = RPW // _CHUNK

    mesh = plsc.VectorSubcoreMesh(core_axis_name="c", subcore_axis_name="s")

    @functools.partial(
        pl.kernel,
        mesh=mesh,
        compiler_params=pltpu.CompilerParams(use_tc_tiling_on_sc=False),
        out_type=jax.ShapeDtypeStruct((R, V), dtype),
        scratch_types=[
            pltpu.VMEM((RPW,), jnp.int32),
            pltpu.VMEM((_CHUNK, V), dtype),
            pltpu.SemaphoreType.DMA,
        ],
    )
    def bigram_gather(table_hbm, src_hbm, out_hbm, idx_v, buf, sem):
        wid = lax.axis_index("s") * _NC + lax.axis_index("c")
        base_row = wid * RPW

        pltpu.sync_copy(src_hbm.at[pl.ds(base_row, RPW)], idx_v)

        def body(c, carry):
            row0 = c * _CHUNK
            pltpu.async_copy(table_hbm.at[idx_v.at[pl.ds(row0, _CHUNK)]],
                             buf, sem).wait()
            pltpu.sync_copy(buf, out_hbm.at[pl.ds(base_row + row0, _CHUNK)])
            return carry

        lax.fori_loop(0, NCH, body, 0)

    return bigram_gather


def kernel(idx, log_probs):
    B, T = idx.shape
    V = log_probs.shape[1]
    Vr = log_probs.shape[0]
    table = jnp.concatenate(
        [log_probs, jnp.zeros((1, V), log_probs.dtype)], axis=0)
    src = jnp.concatenate(
        [jnp.full((B, 1), Vr, jnp.int32), idx[:, :-1].astype(jnp.int32)],
        axis=1).reshape(B * T)
    out_flat = _build(B * T, V, log_probs.dtype.name)(table, src)
    return out_flat.reshape(B, T, V)
